# Initial kernel scaffold; baseline (speedup 1.0000x reference)
import jax
import jax.numpy as jnp
from jax.experimental import pallas as pl


def kernel(x, edge_index, params):
    raise NotImplementedError("write your pallas kernel here")


# speedup vs baseline: 7.0090x; 7.0090x over previous
import functools
import jax
import jax.numpy as jnp
from jax import lax
from jax.experimental import pallas as pl
from jax.experimental.pallas import tpu as pltpu
from jax.experimental.pallas import tpu_sc as plsc

_NC = 2
_NS = 16
_NW = _NC * _NS
_CHUNK = 128

N_FIX = 10000
E_FIX = 160000


@functools.lru_cache(maxsize=None)
def _build_seg_kernel(K, F, n_acc, EP):
    edge_split = (K == 1)
    kpc = 1 if edge_split else K // 2
    n_workers = _NW if edge_split else _NS
    chunks_pw = EP // (n_workers * _CHUNK)
    rpt = n_acc // _NS
    out_shape = (_NC, n_acc, F) if edge_split else (K, n_acc, F)
    mesh = plsc.VectorSubcoreMesh(core_axis_name="c", subcore_axis_name="s")

    @functools.partial(
        pl.kernel, mesh=mesh,
        compiler_params=pltpu.CompilerParams(use_tc_tiling_on_sc=False),
        out_type=jax.ShapeDtypeStruct(out_shape, jnp.float32),
        scratch_types=[
            pltpu.VMEM((_CHUNK,), jnp.int32),
            pltpu.VMEM((_CHUNK,), jnp.int32),
            pltpu.VMEM((kpc, _CHUNK), jnp.float32),
            pltpu.VMEM((_CHUNK, F), jnp.float32),
            pltpu.VMEM((_CHUNK, F), jnp.float32),
            pltpu.VMEM_SHARED((kpc, n_acc, F), jnp.float32),
            pltpu.SemaphoreType.DMA,
        ],
    )
    def k(z_hbm, src_hbm, dst_hbm, w_hbm, zrow_hbm, out_hbm,
          srcv, dstv, wv, rows, srows, acc, sem):
        cid = lax.axis_index("c")
        sid = lax.axis_index("s")
        wid = cid * _NS + sid if edge_split else sid
        kbase = 0 if edge_split else cid * kpc

        for kk in range(kpc):
            pltpu.sync_copy(zrow_hbm, acc.at[kk, pl.ds(sid * rpt, rpt)])
        plsc.subcore_barrier()

        def chunk_body(ci, carry):
            ebase = (wid * chunks_pw + ci) * _CHUNK
            pltpu.sync_copy(src_hbm.at[pl.ds(ebase, _CHUNK)], srcv)
            pltpu.sync_copy(dst_hbm.at[pl.ds(ebase, _CHUNK)], dstv)
            pltpu.sync_copy(w_hbm.at[pl.ds(kbase, kpc), pl.ds(ebase, _CHUNK)], wv)
            pltpu.async_copy(z_hbm.at[srcv], rows, sem).wait()
            for kk in range(kpc):
                def grp_body(g, c2):
                    wvec = wv[kk, pl.ds(g * 16, 16)]
                    for l in range(16):
                        s = wvec[l]
                        e = g * 16 + l
                        for f0 in range(0, F, 16):
                            srows[e, pl.ds(f0, 16)] = rows[e, pl.ds(f0, 16)] * s
                    return c2
                lax.fori_loop(0, _CHUNK // 16, grp_body, 0)
                pltpu.sync_copy(srows, acc.at[kk].at[dstv], add=True)
            return carry

        lax.fori_loop(0, chunks_pw, chunk_body, 0)
        plsc.subcore_barrier()

        for kk in range(kpc):
            pltpu.sync_copy(acc.at[kk, pl.ds(sid * rpt, rpt)],
                            out_hbm.at[kbase + kk, pl.ds(sid * rpt, rpt)])

    return k


def _seg_multi(z, w, srcp, dstp):
    N, F = z.shape
    K, E = w.shape
    kmax = max(2, (88 // F) * 2)
    if K > kmax:
        return jnp.concatenate(
            [_seg_multi(z, w[k0:k0 + kmax], srcp, dstp) for k0 in range(0, K, kmax)])
    EP = ((E + _NW * _CHUNK - 1) // (_NW * _CHUNK)) * (_NW * _CHUNK)
    n_acc = ((N + 1 + _NS * 8 - 1) // (_NS * 8)) * (_NS * 8)
    Kp = K if K == 1 else ((K + 1) // 2) * 2
    wp = jnp.concatenate([w, jnp.zeros((K, EP - E), jnp.float32)], axis=1)
    if Kp != K:
        wp = jnp.concatenate([wp, jnp.zeros((Kp - K, EP), jnp.float32)], axis=0)
    zrow = jnp.zeros((n_acc // _NS, F), jnp.float32)
    fn = _build_seg_kernel(Kp, F, n_acc, EP)
    parts = fn(z, srcp, dstp, wp, zrow)
    if K == 1:
        return (parts[0] + parts[1])[None, :N, :]
    return parts[:K, :N, :]


def _act(h, act):
    if act == 1:
        return jax.nn.sigmoid(h)
    if act == 2:
        return jnp.tanh(h)
    if act == 3:
        return jax.nn.relu(h)
    if act == 4:
        return jax.nn.softmax(h, axis=1)
    return h


@functools.lru_cache(maxsize=None)
def _build_mm(M, Fin, Fout, act, blk):
    def body(xr, wr, br, outr):
        h = jnp.dot(xr[...], wr[...], preferred_element_type=jnp.float32)
        outr[...] = _act(h + br[...], act)

    return pl.pallas_call(
        body,
        grid=(M // blk,),
        in_specs=[
            pl.BlockSpec((blk, Fin), lambda i: (i, 0)),
            pl.BlockSpec((Fin, Fout), lambda i: (0, 0)),
            pl.BlockSpec((1, Fout), lambda i: (0, 0)),
        ],
        out_specs=pl.BlockSpec((blk, Fout), lambda i: (i, 0)),
        out_shape=jax.ShapeDtypeStruct((M, Fout), jnp.float32),
    )


def _mm(x, W, b, act=0):
    M, Fin = x.shape
    Fout = W.shape[1]
    blk = 2000
    fn = _build_mm(M, Fin, Fout, act, blk)
    return fn(x, W, b.reshape(1, -1))


def _forward(x, edge_index, params):
    src = edge_index[0]
    dst = edge_index[1]
    N = x.shape[0]
    E = src.shape[0]
    EP = ((E + _NW * _CHUNK - 1) // (_NW * _CHUNK)) * (_NW * _CHUNK)
    srcp = jnp.concatenate([src, jnp.zeros((EP - E,), jnp.int32)])
    dstp = jnp.concatenate([dst, jnp.full((EP - E,), N, jnp.int32)])

    def seg_multi(z, w):
        return _seg_multi(z, w, srcp, dstp)

    ones_e = jnp.ones((1, E), jnp.float32)
    ones16 = jnp.ones((N, 16), jnp.float32)

    cnt = seg_multi(ones16, ones_e)[0, :, 0]
    degc = jnp.maximum(cnt, 1.0)
    dis = 1.0 / jnp.sqrt(cnt + 1.0)
    sn = dis * dis
    dis_src = dis[src][None, :]
    d1 = dis * seg_multi(ones16, dis_src)[0, :, 0] + sn
    d2 = dis * seg_multi(ones16, (dis * d1)[src][None, :])[0, :, 0] + sn * d1
    d3 = dis * seg_multi(ones16, (dis * d2)[src][None, :])[0, :, 0] + sn * d2
    dcomb = 0.729 * d3 + 0.081 * d2 + 0.09 * d1 + 0.1

    acts_out = [1, 2, 3, 4, 0]
    ta = jax.nn.softmax(params['alpha'])
    h = 0.0
    for i in range(5):
        lin = params['x_lin'][i]
        h = h + ta[i] * _mm(x, lin['W'], lin['b'], acts_out[i])
    ys = [h]

    cache = {}

    def aggs(i):
        if i in cache:
            return cache[i]
        y = ys[i]
        r = seg_multi(y, jnp.concatenate([ones_e, dis_src], axis=0))
        S, Sd = r[0], r[1]
        P1 = dis[:, None] * Sd + sn[:, None] * y
        P2 = dis[:, None] * seg_multi(P1, dis_src)[0] + sn[:, None] * P1
        P3 = dis[:, None] * seg_multi(P2, dis_src)[0] + sn[:, None] * P2
        m = jax.nn.relu(y) + 1e-7
        em = jnp.exp(m - jnp.max(m, axis=0))
        c = dict(S=S, P1=P1, P2=P2, P3=P3,
                 mean=S / degc[:, None], y1=jnp.concatenate([y, jnp.ones((N, 16), jnp.float32)], axis=1))
        G1 = seg_multi(m * em, ones_e)[0]
        G2 = seg_multi(em, ones_e)[0]
        c['gen'] = G1 / (G2 + 1e-16)
        xn = y / (jnp.linalg.norm(y, axis=1, keepdims=True) + 1e-8)
        c['s_agnn'] = jnp.sum(xn[src] * xn[dst], axis=1)
        cache[i] = c
        return c

    for j in range(1, 7):
        beta = params['beta'][str(j)]
        w = jnp.concatenate([jax.nn.softmax(beta[i * 12 + 1:i * 12 + 13])
                             for i in range(j)])
        acc = 0.0
        for i in range(j):
            y = ys[i]
            C = aggs(i)
            convs = params['convs']
            pget = lambda t: convs['%d_%d_%d' % (i, j, t)]

            p1 = pget(1)
            u = y @ (p1['W'] @ p1['a_src'])
            v = y @ (p1['W'] @ p1['a_dst'])
            Kshift = jnp.maximum(jnp.max(u) + jnp.max(v), 0.0)
            E_gat = jnp.exp(jax.nn.leaky_relu(u[src] + v[dst], 0.2) - Kshift)
            p7 = pget(7)
            E_agnn = jnp.exp(p7['beta'] * C['s_agnn'])
            ra = seg_multi(C['y1'], jnp.stack([E_gat, E_agnn]))
            num_gat, den_gat = ra[0, :, :64], ra[0, :, 64] + 1e-16
            num_agnn, den_agnn = ra[1, :, :64], ra[1, :, 64] + 1e-16

            p9 = pget(9)
            g9 = y @ (p9['U'][:, 0] - p9['U'][:, 1])
            q0 = jax.nn.sigmoid(g9[dst] - g9[src] + (p9['c'][0] - p9['c'][1]))
            p11 = pget(11)
            ainv = 1.0 / jnp.sqrt(degc)
            fks = []
            gks = []
            for kk in range(4):
                fks.append(jnp.exp(-0.5 * (ainv - p11['mu'][kk, 0]) ** 2 /
                                   (p11['sigma'][kk, 0] ** 2 + 1e-8))[src])
                gks.append(jnp.exp(-0.5 * (ainv - p11['mu'][kk, 1]) ** 2 /
                                   (p11['sigma'][kk, 1] ** 2 + 1e-8)))
            rb = seg_multi(y, jnp.stack([q0] + fks))
            M0 = rb[0]
            gmm_seg = rb[1:5]

            for t in range(1, 13):
                p = pget(t)
                wt = w[i * 12 + t - 1]
                if t == 1:
                    o = _mm(num_gat / den_gat[:, None], p['W'], p['b'])
                elif t == 2:
                    agg = (1.0 + p['eps']) * y + C['S']
                    hh = _mm(agg, p['W1'], p['b1'])
                    hh = (hh - jnp.mean(hh, 0)) / jnp.sqrt(jnp.var(hh, 0) + 1e-5) * p['bn_g'] + p['bn_b']
                    o = _mm(jax.nn.relu(hh), p['W2'], p['b2'])
                elif t == 3:
                    o = _mm(y, p['Wr'], p['b']) + _mm(C['mean'], p['Wn'], jnp.zeros((64,), jnp.float32))
                elif t == 4:
                    o = _mm(C['P1'], p['W'], p['b'])
                elif t == 5:
                    o = _mm(C['P2'], p['W'], p['b'])
                elif t == 6:
                    ycomb = 0.729 * C['P3'] + 0.081 * C['P2'] + 0.09 * C['P1'] + 0.1 * y
                    o = _mm(ycomb, p['W'], jnp.zeros((64,), jnp.float32)) + dcomb[:, None] * p['b'][None, :]
                elif t == 7:
                    o = num_agnn / den_agnn[:, None]
                elif t == 8:
                    o = jax.nn.relu(_mm(C['P1'], p['Wi'], p['b']) + _mm(y, p['V'], jnp.zeros((64,), jnp.float32)))
                elif t == 9:
                    o = (_mm(M0, p['W'][0], jnp.zeros((64,), jnp.float32))
                         + _mm(C['S'] - M0, p['W'][1], jnp.zeros((64,), jnp.float32))) / degc[:, None] + p['b']
                elif t == 10:
                    hh = _mm(y + C['gen'], p['W1'], p['b1'], 3)
                    o = _mm(hh, p['W2'], p['b2'])
                elif t == 11:
                    o = 0.0
                    for kk in range(4):
                        o = o + gks[kk][:, None] * _mm(gmm_seg[kk], p['W'][kk], jnp.zeros((64,), jnp.float32))
                    o = o / degc[:, None] + p['b']
                else:
                    hh = y
                    for l in range(3):
                        Sh = C['S'] if l == 0 else seg_multi(hh, ones_e)[0]
                        mm_ = _mm(Sh, p['Wl'][l], p['bl'][l])
                        r_ = jax.nn.sigmoid(_mm(mm_, p['Wir'], p['br']) + _mm(hh, p['Whr'], jnp.zeros((64,), jnp.float32)))
                        z_ = jax.nn.sigmoid(_mm(mm_, p['Wiz'], p['bz']) + _mm(hh, p['Whz'], jnp.zeros((64,), jnp.float32)))
                        n_ = jnp.tanh(_mm(mm_, p['Win'], p['bn']) + r_ * _mm(hh, p['Whn'], jnp.zeros((64,), jnp.float32)))
                        hh = (1.0 - z_) * n_ + z_ * hh
                    o = hh
                acc = acc + wt * o
        ys.append(acc)

    xs = ys[1] + ys[2] + ys[3] + ys[4] + ys[5] + ys[6]
    tg = jax.nn.softmax(params['gamma'])
    out = 0.0
    for i in range(5):
        lin = params['z_lin'][i]
        out = out + tg[i] * _mm(xs, lin['W'], lin['b'], acts_out[i])
    return out


def kernel(x, edge_index, params):
    return _forward(x, edge_index, params)


# speedup vs baseline: 7.3223x; 1.0447x over previous
import functools
import jax
import jax.numpy as jnp
from jax import lax
from jax.experimental import pallas as pl
from jax.experimental.pallas import tpu as pltpu
from jax.experimental.pallas import tpu_sc as plsc

_NC = 2
_NS = 16
_NW = _NC * _NS
_CHUNK = 128

N_FIX = 10000
E_FIX = 160000


@functools.lru_cache(maxsize=None)
def _build_seg_kernel(K, F, n_acc, EP):
    edge_split = (K == 1)
    kpc = 1 if edge_split else K // 2
    n_workers = _NW if edge_split else _NS
    chunks_pw = EP // (n_workers * _CHUNK)
    rpt = n_acc // _NS
    out_shape = (_NC, n_acc, F) if edge_split else (K, n_acc, F)
    mesh = plsc.VectorSubcoreMesh(core_axis_name="c", subcore_axis_name="s")

    @functools.partial(
        pl.kernel, mesh=mesh,
        compiler_params=pltpu.CompilerParams(use_tc_tiling_on_sc=False),
        out_type=jax.ShapeDtypeStruct(out_shape, jnp.float32),
        scratch_types=[
            pltpu.VMEM((chunks_pw, _CHUNK), jnp.int32),
            pltpu.VMEM((chunks_pw, _CHUNK), jnp.int32),
            pltpu.VMEM((kpc, chunks_pw, _CHUNK), jnp.float32),
            pltpu.VMEM((2, _CHUNK, F), jnp.float32),
            pltpu.VMEM((_CHUNK, F), jnp.float32),
            pltpu.VMEM_SHARED((kpc, n_acc, F), jnp.float32),
            pltpu.SemaphoreType.DMA,
        ],
    )
    def k(z_hbm, src_hbm, dst_hbm, w_hbm, zrow_hbm, out_hbm,
          srcs, dsts, ws, rows, srows, acc, sem):
        cid = lax.axis_index("c")
        sid = lax.axis_index("s")
        wid = cid * _NS + sid if edge_split else sid
        kbase = 0 if edge_split else cid * kpc
        row0 = wid * chunks_pw

        pltpu.sync_copy(src_hbm.at[pl.ds(row0, chunks_pw)], srcs)
        pltpu.sync_copy(dst_hbm.at[pl.ds(row0, chunks_pw)], dsts)
        pltpu.sync_copy(w_hbm.at[pl.ds(kbase, kpc), pl.ds(row0, chunks_pw)], ws)
        for kk in range(kpc):
            pltpu.sync_copy(zrow_hbm, acc.at[kk, pl.ds(sid * rpt, rpt)])
        plsc.subcore_barrier()

        pltpu.async_copy(z_hbm.at[srcs.at[0]], rows.at[0], sem)

        def pair_body(g, carry):
            for b in range(2):
                ci = g * 2 + b
                nci = lax.rem(ci + 1, chunks_pw)
                pltpu.async_copy(z_hbm.at[srcs.at[nci]], rows.at[1 - b], sem)
                pltpu.make_async_copy(z_hbm.at[srcs.at[ci]], rows.at[b], sem).wait()
                for kk in range(kpc):
                    def grp_body(g16, c2):
                        wvec = ws[kk, ci, pl.ds(g16 * 16, 16)]
                        for l in range(16):
                            s = wvec[l]
                            e = g16 * 16 + l
                            for f0 in range(0, F, 16):
                                srows[e, pl.ds(f0, 16)] = rows[b, e, pl.ds(f0, 16)] * s
                        return c2
                    lax.fori_loop(0, _CHUNK // 16, grp_body, 0)
                    pltpu.sync_copy(srows, acc.at[kk].at[dsts.at[ci]], add=True)
            return carry

        lax.fori_loop(0, chunks_pw // 2, pair_body, 0)
        pltpu.make_async_copy(z_hbm.at[srcs.at[0]], rows.at[0], sem).wait()
        plsc.subcore_barrier()

        for kk in range(kpc):
            pltpu.sync_copy(acc.at[kk, pl.ds(sid * rpt, rpt)],
                            out_hbm.at[kbase + kk, pl.ds(sid * rpt, rpt)])

    return k


def _seg_multi(z, w, srcp, dstp):
    N, F = z.shape
    K, E = w.shape
    kmax = max(2, (88 // F) * 2)
    if K > kmax:
        return jnp.concatenate(
            [_seg_multi(z, w[k0:k0 + kmax], srcp, dstp) for k0 in range(0, K, kmax)])
    EP = ((E + _NW * _CHUNK - 1) // (_NW * _CHUNK)) * (_NW * _CHUNK)
    n_acc = ((N + 1 + _NS * 8 - 1) // (_NS * 8)) * (_NS * 8)
    Kp = K if K == 1 else ((K + 1) // 2) * 2
    wp = jnp.concatenate([w, jnp.zeros((K, EP - E), jnp.float32)], axis=1)
    if Kp != K:
        wp = jnp.concatenate([wp, jnp.zeros((Kp - K, EP), jnp.float32)], axis=0)
    zrow = jnp.zeros((n_acc // _NS, F), jnp.float32)
    fn = _build_seg_kernel(Kp, F, n_acc, EP)
    parts = fn(z, srcp.reshape(-1, _CHUNK), dstp.reshape(-1, _CHUNK),
               wp.reshape(Kp, -1, _CHUNK), zrow)
    if K == 1:
        return (parts[0] + parts[1])[None, :N, :]
    return parts[:K, :N, :]


def _act(h, act):
    if act == 1:
        return jax.nn.sigmoid(h)
    if act == 2:
        return jnp.tanh(h)
    if act == 3:
        return jax.nn.relu(h)
    if act == 4:
        return jax.nn.softmax(h, axis=1)
    return h


@functools.lru_cache(maxsize=None)
def _build_mm(M, Fin, Fout, act, blk):
    def body(xr, wr, br, outr):
        h = jnp.dot(xr[...], wr[...], preferred_element_type=jnp.float32)
        outr[...] = _act(h + br[...], act)

    return pl.pallas_call(
        body,
        grid=(M // blk,),
        in_specs=[
            pl.BlockSpec((blk, Fin), lambda i: (i, 0)),
            pl.BlockSpec((Fin, Fout), lambda i: (0, 0)),
            pl.BlockSpec((1, Fout), lambda i: (0, 0)),
        ],
        out_specs=pl.BlockSpec((blk, Fout), lambda i: (i, 0)),
        out_shape=jax.ShapeDtypeStruct((M, Fout), jnp.float32),
    )


def _mm(x, W, b, act=0):
    M, Fin = x.shape
    Fout = W.shape[1]
    blk = 2000
    fn = _build_mm(M, Fin, Fout, act, blk)
    return fn(x, W, b.reshape(1, -1))


def _forward(x, edge_index, params):
    src = edge_index[0]
    dst = edge_index[1]
    N = x.shape[0]
    E = src.shape[0]
    EP = ((E + _NW * _CHUNK - 1) // (_NW * _CHUNK)) * (_NW * _CHUNK)
    srcp = jnp.concatenate([src, jnp.zeros((EP - E,), jnp.int32)])
    dstp = jnp.concatenate([dst, jnp.full((EP - E,), N, jnp.int32)])

    def seg_multi(z, w):
        return _seg_multi(z, w, srcp, dstp)

    ones_e = jnp.ones((1, E), jnp.float32)
    ones16 = jnp.ones((N, 16), jnp.float32)

    cnt = seg_multi(ones16, ones_e)[0, :, 0]
    degc = jnp.maximum(cnt, 1.0)
    dis = 1.0 / jnp.sqrt(cnt + 1.0)
    sn = dis * dis
    dis_src = dis[src][None, :]
    d1 = dis * seg_multi(ones16, dis_src)[0, :, 0] + sn
    d2 = dis * seg_multi(ones16, (dis * d1)[src][None, :])[0, :, 0] + sn * d1
    d3 = dis * seg_multi(ones16, (dis * d2)[src][None, :])[0, :, 0] + sn * d2
    dcomb = 0.729 * d3 + 0.081 * d2 + 0.09 * d1 + 0.1

    acts_out = [1, 2, 3, 4, 0]
    ta = jax.nn.softmax(params['alpha'])
    h = 0.0
    for i in range(5):
        lin = params['x_lin'][i]
        h = h + ta[i] * _mm(x, lin['W'], lin['b'], acts_out[i])
    ys = [h]

    cache = {}

    def aggs(i):
        if i in cache:
            return cache[i]
        y = ys[i]
        r = seg_multi(y, jnp.concatenate([ones_e, dis_src], axis=0))
        S, Sd = r[0], r[1]
        P1 = dis[:, None] * Sd + sn[:, None] * y
        P2 = dis[:, None] * seg_multi(P1, dis_src)[0] + sn[:, None] * P1
        P3 = dis[:, None] * seg_multi(P2, dis_src)[0] + sn[:, None] * P2
        m = jax.nn.relu(y) + 1e-7
        em = jnp.exp(m - jnp.max(m, axis=0))
        c = dict(S=S, P1=P1, P2=P2, P3=P3,
                 mean=S / degc[:, None], y1=jnp.concatenate([y, jnp.ones((N, 16), jnp.float32)], axis=1))
        G1 = seg_multi(m * em, ones_e)[0]
        G2 = seg_multi(em, ones_e)[0]
        c['gen'] = G1 / (G2 + 1e-16)
        xn = y / (jnp.linalg.norm(y, axis=1, keepdims=True) + 1e-8)
        c['s_agnn'] = jnp.sum(xn[src] * xn[dst], axis=1)
        cache[i] = c
        return c

    for j in range(1, 7):
        beta = params['beta'][str(j)]
        w = jnp.concatenate([jax.nn.softmax(beta[i * 12 + 1:i * 12 + 13])
                             for i in range(j)])
        acc = 0.0
        for i in range(j):
            y = ys[i]
            C = aggs(i)
            convs = params['convs']
            pget = lambda t: convs['%d_%d_%d' % (i, j, t)]

            p1 = pget(1)
            p9 = pget(9)
            uvg = _mm(y, jnp.stack([p1['W'] @ p1['a_src'], p1['W'] @ p1['a_dst'],
                                    p9['U'][:, 0] - p9['U'][:, 1]], axis=1),
                      jnp.zeros((3,), jnp.float32))
            u, v, g9 = uvg[:, 0], uvg[:, 1], uvg[:, 2]
            Kshift = jnp.maximum(jnp.max(u) + jnp.max(v), 0.0)
            E_gat = jnp.exp(jax.nn.leaky_relu(u[src] + v[dst], 0.2) - Kshift)
            p7 = pget(7)
            E_agnn = jnp.exp(p7['beta'] * C['s_agnn'])
            ra = seg_multi(C['y1'], jnp.stack([E_gat, E_agnn]))
            num_gat, den_gat = ra[0, :, :64], ra[0, :, 64] + 1e-16
            num_agnn, den_agnn = ra[1, :, :64], ra[1, :, 64] + 1e-16

            q0 = jax.nn.sigmoid(g9[dst] - g9[src] + (p9['c'][0] - p9['c'][1]))
            p11 = pget(11)
            ainv = 1.0 / jnp.sqrt(degc)
            fks = []
            gks = []
            for kk in range(4):
                fks.append(jnp.exp(-0.5 * (ainv - p11['mu'][kk, 0]) ** 2 /
                                   (p11['sigma'][kk, 0] ** 2 + 1e-8))[src])
                gks.append(jnp.exp(-0.5 * (ainv - p11['mu'][kk, 1]) ** 2 /
                                   (p11['sigma'][kk, 1] ** 2 + 1e-8)))
            rb = seg_multi(y, jnp.stack([q0] + fks))
            M0 = rb[0]
            gmm_seg = rb[1:5]

            for t in range(1, 13):
                p = pget(t)
                wt = w[i * 12 + t - 1]
                if t == 1:
                    o = _mm(num_gat / den_gat[:, None], p['W'], p['b'])
                elif t == 2:
                    agg = (1.0 + p['eps']) * y + C['S']
                    hh = _mm(agg, p['W1'], p['b1'])
                    hh = (hh - jnp.mean(hh, 0)) / jnp.sqrt(jnp.var(hh, 0) + 1e-5) * p['bn_g'] + p['bn_b']
                    o = _mm(jax.nn.relu(hh), p['W2'], p['b2'])
                elif t == 3:
                    o = _mm(jnp.concatenate([y, C['mean']], axis=1),
                            jnp.concatenate([p['Wr'], p['Wn']], axis=0), p['b'])
                elif t == 4:
                    o = _mm(C['P1'], p['W'], p['b'])
                elif t == 5:
                    o = _mm(C['P2'], p['W'], p['b'])
                elif t == 6:
                    ycomb = 0.729 * C['P3'] + 0.081 * C['P2'] + 0.09 * C['P1'] + 0.1 * y
                    o = _mm(ycomb, p['W'], jnp.zeros((64,), jnp.float32)) + dcomb[:, None] * p['b'][None, :]
                elif t == 7:
                    o = num_agnn / den_agnn[:, None]
                elif t == 8:
                    o = _mm(jnp.concatenate([C['P1'], y], axis=1),
                            jnp.concatenate([p['Wi'], p['V']], axis=0), p['b'], 3)
                elif t == 9:
                    o = _mm(jnp.concatenate([M0, C['S'] - M0], axis=1),
                            p['W'].reshape(128, 64),
                            jnp.zeros((64,), jnp.float32)) / degc[:, None] + p['b']
                elif t == 10:
                    hh = _mm(y + C['gen'], p['W1'], p['b1'], 3)
                    o = _mm(hh, p['W2'], p['b2'])
                elif t == 11:
                    zin = jnp.concatenate(
                        [gks[kk][:, None] * gmm_seg[kk] for kk in range(4)], axis=1)
                    o = _mm(zin, p['W'].reshape(256, 64),
                            jnp.zeros((64,), jnp.float32)) / degc[:, None] + p['b']
                else:
                    Wi3 = jnp.concatenate([p['Wir'], p['Wiz'], p['Win']], axis=1)
                    bi3 = jnp.concatenate([p['br'], p['bz'], p['bn']])
                    Wh3 = jnp.concatenate([p['Whr'], p['Whz'], p['Whn']], axis=1)
                    hh = y
                    for l in range(3):
                        Sh = C['S'] if l == 0 else seg_multi(hh, ones_e)[0]
                        mm_ = _mm(Sh, p['Wl'][l], p['bl'][l])
                        m3 = _mm(mm_, Wi3, bi3)
                        h3 = _mm(hh, Wh3, jnp.zeros((192,), jnp.float32))
                        r_ = jax.nn.sigmoid(m3[:, 0:64] + h3[:, 0:64])
                        z_ = jax.nn.sigmoid(m3[:, 64:128] + h3[:, 64:128])
                        n_ = jnp.tanh(m3[:, 128:192] + r_ * h3[:, 128:192])
                        hh = (1.0 - z_) * n_ + z_ * hh
                    o = hh
                acc = acc + wt * o
        ys.append(acc)

    xs = ys[1] + ys[2] + ys[3] + ys[4] + ys[5] + ys[6]
    tg = jax.nn.softmax(params['gamma'])
    out = 0.0
    for i in range(5):
        lin = params['z_lin'][i]
        out = out + tg[i] * _mm(xs, lin['W'], lin['b'], acts_out[i])
    return out


def kernel(x, edge_index, params):
    return _forward(x, edge_index, params)
